# baseline (speedup 1.0000x reference)
import jax
import jax.numpy as jnp
from jax.experimental import pallas as pl
from jax.experimental.pallas import tpu as pltpu

_B, _T, _C = 16, 32768, 64
_TB = 8192


def _loss_body(p_ref, t_ref, m_ref, se_ref, n_ref):
    i = pl.program_id(0)
    j = pl.program_id(1)

    @pl.when((i == 0) & (j == 0))
    def _():
        se_ref[...] = jnp.zeros_like(se_ref)
        n_ref[...] = jnp.zeros_like(n_ref)

    d = p_ref[0] - t_ref[0]
    m = m_ref[0].astype(jnp.float32)
    se_ref[...] += jnp.sum(d * d * m, keepdims=True)
    n_ref[...] += jnp.sum(m, keepdims=True)


def kernel(predicted, target, mask):
    m3 = mask.reshape(_B, _T, 1)

    grid = (_B, _T // _TB)
    se_sum, n_sum = pl.pallas_call(
        _loss_body,
        grid=grid,
        in_specs=[
            pl.BlockSpec((1, _TB, _C), lambda i, j: (i, j, 0)),
            pl.BlockSpec((1, _TB, _C), lambda i, j: (i, j, 0)),
            pl.BlockSpec((1, _TB, 1), lambda i, j: (i, j, 0)),
        ],
        out_specs=[
            pl.BlockSpec((1, 1), lambda i, j: (0, 0)),
            pl.BlockSpec((1, 1), lambda i, j: (0, 0)),
        ],
        out_shape=[
            jax.ShapeDtypeStruct((1, 1), jnp.float32),
            jax.ShapeDtypeStruct((1, 1), jnp.float32),
        ],
        compiler_params=pltpu.CompilerParams(
            dimension_semantics=("arbitrary", "arbitrary"),
        ),
    )(predicted, target, m3)

    se = se_sum[0, 0]
    n = n_sum[0, 0]
    count = n * jnp.float32(_C)
    safe = jnp.where(count == 0.0, jnp.float32(1.0), count)
    return jnp.where(n == 0.0, jnp.float32(0.0), se / safe)


# speedup vs baseline: 1.2019x; 1.2019x over previous
import jax
import jax.numpy as jnp
from jax.experimental import pallas as pl
from jax.experimental.pallas import tpu as pltpu

_B, _T, _C = 16, 32768, 64
_TB = 8192
_MR = _TB // 128


def _loss_body(p_ref, t_ref, m_ref, se_ref, n_ref, se_acc, n_acc):
    i = pl.program_id(0)
    j = pl.program_id(1)

    @pl.when((i == 0) & (j == 0))
    def _():
        se_acc[0] = 0.0
        n_acc[0] = 0.0

    d = p_ref[0] - t_ref[0]
    rs = jnp.sum(d * d, axis=1)
    mf = m_ref[0].astype(jnp.float32)
    mflat = mf.reshape(_TB)
    se_acc[0] += jnp.sum(rs * mflat)
    n_acc[0] += jnp.sum(mf)

    @pl.when((i == _B - 1) & (j == (_T // _TB) - 1))
    def _():
        se_ref[0] = se_acc[0]
        n_ref[0] = n_acc[0]


def kernel(predicted, target, mask):
    m3 = mask.reshape(_B, _T // 128, 128)

    grid = (_B, _T // _TB)
    se_sum, n_sum = pl.pallas_call(
        _loss_body,
        grid=grid,
        in_specs=[
            pl.BlockSpec((1, _TB, _C), lambda i, j: (i, j, 0)),
            pl.BlockSpec((1, _TB, _C), lambda i, j: (i, j, 0)),
            pl.BlockSpec((1, _MR, 128), lambda i, j: (i, j, 0)),
        ],
        out_specs=[
            pl.BlockSpec(memory_space=pltpu.MemorySpace.SMEM),
            pl.BlockSpec(memory_space=pltpu.MemorySpace.SMEM),
        ],
        out_shape=[
            jax.ShapeDtypeStruct((1,), jnp.float32),
            jax.ShapeDtypeStruct((1,), jnp.float32),
        ],
        scratch_shapes=[
            pltpu.SMEM((1,), jnp.float32),
            pltpu.SMEM((1,), jnp.float32),
        ],
        compiler_params=pltpu.CompilerParams(
            dimension_semantics=("arbitrary", "arbitrary"),
        ),
    )(predicted, target, m3)

    se = se_sum[0]
    n = n_sum[0]
    count = n * jnp.float32(_C)
    safe = jnp.where(count == 0.0, jnp.float32(1.0), count)
    return jnp.where(n == 0.0, jnp.float32(0.0), se / safe)


# speedup vs baseline: 1.2246x; 1.0189x over previous
import jax
import jax.numpy as jnp
from jax.experimental import pallas as pl
from jax.experimental.pallas import tpu as pltpu

_B, _T, _C = 16, 32768, 64
_TB = 16384
_MR = _TB // 128


def _loss_body(p_ref, t_ref, m_ref, se_ref, n_ref, se_acc, n_acc):
    i = pl.program_id(0)
    j = pl.program_id(1)

    @pl.when((i == 0) & (j == 0))
    def _():
        se_acc[0] = 0.0
        n_acc[0] = 0.0

    d = p_ref[0] - t_ref[0]
    rs = jnp.sum(d * d, axis=1)
    mf = m_ref[0].astype(jnp.float32)
    mflat = mf.reshape(_TB)
    se_acc[0] += jnp.sum(rs * mflat)
    n_acc[0] += jnp.sum(mf)

    @pl.when((i == _B - 1) & (j == (_T // _TB) - 1))
    def _():
        se_ref[0] = se_acc[0]
        n_ref[0] = n_acc[0]


def kernel(predicted, target, mask):
    m3 = mask.reshape(_B, _T // 128, 128)

    grid = (_B, _T // _TB)
    se_sum, n_sum = pl.pallas_call(
        _loss_body,
        grid=grid,
        in_specs=[
            pl.BlockSpec((1, _TB, _C), lambda i, j: (i, j, 0)),
            pl.BlockSpec((1, _TB, _C), lambda i, j: (i, j, 0)),
            pl.BlockSpec((1, _MR, 128), lambda i, j: (i, j, 0)),
        ],
        out_specs=[
            pl.BlockSpec(memory_space=pltpu.MemorySpace.SMEM),
            pl.BlockSpec(memory_space=pltpu.MemorySpace.SMEM),
        ],
        out_shape=[
            jax.ShapeDtypeStruct((1,), jnp.float32),
            jax.ShapeDtypeStruct((1,), jnp.float32),
        ],
        scratch_shapes=[
            pltpu.SMEM((1,), jnp.float32),
            pltpu.SMEM((1,), jnp.float32),
        ],
        compiler_params=pltpu.CompilerParams(
            dimension_semantics=("arbitrary", "arbitrary"),
            vmem_limit_bytes=60 * 1024 * 1024,
        ),
    )(predicted, target, m3)

    se = se_sum[0]
    n = n_sum[0]
    count = n * jnp.float32(_C)
    safe = jnp.where(count == 0.0, jnp.float32(1.0), count)
    return jnp.where(n == 0.0, jnp.float32(0.0), se / safe)


# speedup vs baseline: 1.3710x; 1.1196x over previous
import jax
import jax.numpy as jnp
from jax.experimental import pallas as pl
from jax.experimental.pallas import tpu as pltpu

_B, _T, _C = 16, 32768, 64
_TB = 8192
_NQ = 4
_CH = _TB // _NQ
_SLABS_PER_B = _T // _TB
_STEPS = _B * _SLABS_PER_B
_MR = _TB // 128


def _loss_body(p_hbm, t_hbm, m_ref, se_ref, n_ref,
               pb, tb, se_acc, n_acc, sp, st):
    i = pl.program_id(0)

    def fire(j, h):
        b = j // _SLABS_PER_B
        t0 = (j % _SLABS_PER_B) * _TB
        for q in range(_NQ):
            pltpu.make_async_copy(
                p_hbm.at[b, pl.ds(t0 + q * _CH, _CH)],
                pb.at[h, pl.ds(q * _CH, _CH)],
                sp.at[h, q],
            ).start()
            pltpu.make_async_copy(
                t_hbm.at[b, pl.ds(t0 + q * _CH, _CH)],
                tb.at[h, pl.ds(q * _CH, _CH)],
                st.at[h, q],
            ).start()

    def wait(j, h):
        b = j // _SLABS_PER_B
        t0 = (j % _SLABS_PER_B) * _TB
        for q in range(_NQ):
            pltpu.make_async_copy(
                p_hbm.at[b, pl.ds(t0 + q * _CH, _CH)],
                pb.at[h, pl.ds(q * _CH, _CH)],
                sp.at[h, q],
            ).wait()
            pltpu.make_async_copy(
                t_hbm.at[b, pl.ds(t0 + q * _CH, _CH)],
                tb.at[h, pl.ds(q * _CH, _CH)],
                st.at[h, q],
            ).wait()

    @pl.when(i == 0)
    def _():
        se_acc[0] = 0.0
        n_acc[0] = 0.0
        fire(0, 0)

    for h in (0, 1):
        @pl.when(((i + 1) % 2 == h) & (i + 1 < _STEPS))
        def _():
            fire(i + 1, h)

    for h in (0, 1):
        @pl.when(i % 2 == h)
        def _():
            wait(i, h)
            d = pb[h] - tb[h]
            d2 = (d * d).astype(jnp.bfloat16)
            mrow = m_ref[0].astype(jnp.bfloat16).reshape(1, _TB)
            part = jax.lax.dot_general(
                mrow, d2, (((1,), (0,)), ((), ())),
                preferred_element_type=jnp.float32)
            se_acc[0] += jnp.sum(part)
            n_acc[0] += jnp.sum(m_ref[0].astype(jnp.float32))

    @pl.when(i == _STEPS - 1)
    def _():
        se_ref[0] = se_acc[0]
        n_ref[0] = n_acc[0]


def kernel(predicted, target, mask):
    m3 = mask.reshape(_B, _T // 128, 128)

    grid = (_STEPS,)
    se_sum, n_sum = pl.pallas_call(
        _loss_body,
        grid=grid,
        in_specs=[
            pl.BlockSpec(memory_space=pltpu.MemorySpace.HBM),
            pl.BlockSpec(memory_space=pltpu.MemorySpace.HBM),
            pl.BlockSpec((1, _MR, 128),
                         lambda i: (i // _SLABS_PER_B, i % _SLABS_PER_B, 0)),
        ],
        out_specs=[
            pl.BlockSpec(memory_space=pltpu.MemorySpace.SMEM),
            pl.BlockSpec(memory_space=pltpu.MemorySpace.SMEM),
        ],
        out_shape=[
            jax.ShapeDtypeStruct((1,), jnp.float32),
            jax.ShapeDtypeStruct((1,), jnp.float32),
        ],
        scratch_shapes=[
            pltpu.VMEM((2, _TB, _C), jnp.float32),
            pltpu.VMEM((2, _TB, _C), jnp.float32),
            pltpu.SMEM((1,), jnp.float32),
            pltpu.SMEM((1,), jnp.float32),
            pltpu.SemaphoreType.DMA((2, _NQ)),
            pltpu.SemaphoreType.DMA((2, _NQ)),
        ],
        compiler_params=pltpu.CompilerParams(
            dimension_semantics=("arbitrary",),
            vmem_limit_bytes=60 * 1024 * 1024,
        ),
    )(predicted, target, m3)

    se = se_sum[0]
    n = n_sum[0]
    count = n * jnp.float32(_C)
    safe = jnp.where(count == 0.0, jnp.float32(1.0), count)
    return jnp.where(n == 0.0, jnp.float32(0.0), se / safe)
